# Initial kernel scaffold; baseline (speedup 1.0000x reference)
import jax
import jax.numpy as jnp
from jax.experimental import pallas as pl


def kernel(x, edge_index, batch, target_embedding, c1_w1, c1_b1, c1_w2, c1_b2, cw1, cb1, cw2, cb2, bn_g, bn_b, fcxd_w, fcxd_b, fcxt_w, fcxt_b, bnxt_g, bnxt_b, fc1_w, fc1_b, fc2_w, fc2_b, out_w, out_b):
    raise NotImplementedError("write your pallas kernel here")


# speedup vs baseline: 3.3489x; 3.3489x over previous
import functools

import jax
import jax.numpy as jnp
from jax import lax
from jax.experimental import pallas as pl
from jax.experimental.pallas import tpu as pltpu
from jax.experimental.pallas import tpu_sc as plsc

N = 50000
E = 800000
B = 256
H = 32
F = 78
FPAD = 80

NC = 2
NS = 16
NW = NC * NS

EBLK = 128
NPAD = 53248
EPAD = 819200
E_TBLK = EPAD // (NW * EBLK)
ECH = 8
N_TROW = NPAD // NS
P_TBLK = NPAD // (NW * EBLK)
P_TPAD = 16
BACC = 384
B_TROW = BACC // NS

TCBLK = 512
NGRID = NPAD // TCBLK


def _sc_mesh():
    return plsc.VectorSubcoreMesh(core_axis_name="c", subcore_axis_name="s",
                                  num_cores=NC, num_subcores=NS)


@functools.cache
def _make_sc_edge_agg(w):
  @functools.partial(
    pl.kernel,
    out_type=jax.ShapeDtypeStruct((NC * NPAD, w), jnp.float32),
    mesh=_sc_mesh(),
    scratch_types=[
        pltpu.VMEM((ECH, EBLK), jnp.int32),
        pltpu.VMEM((ECH, EBLK), jnp.int32),
        pltpu.VMEM((EBLK, w), jnp.float32),
        pltpu.VMEM_SHARED((NPAD, w), jnp.float32),
        pltpu.SemaphoreType.DMA,
    ],
    compiler_params=pltpu.CompilerParams(use_tc_tiling_on_sc=False),
  )
  def _sc_edge_agg(y_hbm, src_hbm, dst_hbm, zero_hbm, out_hbm,
                   src_v, dst_v, rows_v, acc_sh, sem):
    cid = lax.axis_index("c")
    sid = lax.axis_index("s")
    wid = sid * NC + cid

    pltpu.sync_copy(zero_hbm, acc_sh.at[pl.ds(sid * N_TROW, N_TROW)])
    plsc.subcore_barrier()

    def body(c, carry):
        base = wid * E_TBLK + c * ECH
        pltpu.sync_copy(src_hbm.at[pl.ds(base, ECH)], src_v)
        pltpu.sync_copy(dst_hbm.at[pl.ds(base, ECH)], dst_v)
        for j in range(ECH):
            pltpu.async_copy(y_hbm.at[src_v.at[j]], rows_v, sem).wait()
            pltpu.sync_copy(rows_v, acc_sh.at[dst_v.at[j]], add=True)
        return carry

    lax.fori_loop(0, E_TBLK // ECH, body, 0)
    plsc.subcore_barrier()

    pltpu.sync_copy(acc_sh.at[pl.ds(sid * N_TROW, N_TROW)],
                    out_hbm.at[pl.ds(cid * NPAD + sid * N_TROW, N_TROW)])

  return _sc_edge_agg


@functools.cache
def _make_sc_pool():
  @functools.partial(
    pl.kernel,
    out_type=jax.ShapeDtypeStruct((NC * BACC, H), jnp.float32),
    mesh=_sc_mesh(),
    scratch_types=[
        pltpu.VMEM((P_TPAD, EBLK), jnp.int32),
        pltpu.VMEM((EBLK, H), jnp.float32),
        pltpu.VMEM_SHARED((BACC, H), jnp.float32),
        pltpu.SemaphoreType.DMA,
    ],
    compiler_params=pltpu.CompilerParams(use_tc_tiling_on_sc=False),
  )
  def _sc_pool(h_hbm, bid_hbm, zero_hbm, out_hbm, bid_v, rows_v, acc_sh, sem):
    cid = lax.axis_index("c")
    sid = lax.axis_index("s")
    wid = sid * NC + cid

    pltpu.sync_copy(zero_hbm.at[pl.ds(0, B_TROW)],
                    acc_sh.at[pl.ds(sid * B_TROW, B_TROW)])
    pltpu.sync_copy(bid_hbm.at[wid], bid_v)
    plsc.subcore_barrier()

    def body(j, carry):
        pltpu.async_copy(
            h_hbm.at[pl.ds((wid * P_TBLK + j) * EBLK, EBLK)], rows_v,
            sem).wait()
        pltpu.sync_copy(rows_v, acc_sh.at[bid_v.at[j]], add=True)
        return carry

    lax.fori_loop(0, P_TBLK, body, 0)
    plsc.subcore_barrier()

    pltpu.sync_copy(acc_sh.at[pl.ds(sid * B_TROW, B_TROW)],
                    out_hbm.at[pl.ds(cid * BACC + sid * B_TROW, B_TROW)])

  return _sc_pool


def _gin_body(h_ref, a0_ref, a1_ref, w1_ref, b1_ref, w2_ref, b2_ref,
              hp_ref, s_ref):
    i = pl.program_id(0)
    m = h_ref[...] + (a0_ref[...] + a1_ref[...])
    u = jnp.maximum(
        jnp.dot(m, w1_ref[...], preferred_element_type=jnp.float32)
        + b1_ref[...], 0.0)
    hp = jnp.maximum(
        jnp.dot(u, w2_ref[...], preferred_element_type=jnp.float32)
        + b2_ref[...], 0.0)
    hp_ref[...] = hp
    row = i * TCBLK + lax.broadcasted_iota(jnp.int32, (TCBLK, 1), 0)
    hm = jnp.where(row < N, hp, 0.0)

    @pl.when(i == 0)
    def _():
        s_ref[...] = jnp.zeros_like(s_ref)

    s_ref[...] += jnp.sum(hm, axis=0, keepdims=True)


def _tc_gin(h, a0, a1, w1, b1, w2, b2):
    fin = h.shape[1]
    return pl.pallas_call(
        _gin_body,
        grid=(NGRID,),
        in_specs=[
            pl.BlockSpec((TCBLK, fin), lambda i: (i, 0)),
            pl.BlockSpec((TCBLK, fin), lambda i: (i, 0)),
            pl.BlockSpec((TCBLK, fin), lambda i: (i, 0)),
            pl.BlockSpec((fin, H), lambda i: (0, 0)),
            pl.BlockSpec((1, H), lambda i: (0, 0)),
            pl.BlockSpec((H, H), lambda i: (0, 0)),
            pl.BlockSpec((1, H), lambda i: (0, 0)),
        ],
        out_specs=[
            pl.BlockSpec((TCBLK, H), lambda i: (i, 0)),
            pl.BlockSpec((1, H), lambda i: (0, 0)),
        ],
        out_shape=[
            jax.ShapeDtypeStruct((NPAD, H), jnp.float32),
            jax.ShapeDtypeStruct((1, H), jnp.float32),
        ],
    )(h, a0, a1, w1, b1, w2, b2)


def _ssd_body(hp_ref, mu_ref, ssd_ref):
    i = pl.program_id(0)
    row = i * TCBLK + lax.broadcasted_iota(jnp.int32, (TCBLK, 1), 0)
    d = hp_ref[...] - mu_ref[...]
    d = jnp.where(row < N, d, 0.0)

    @pl.when(i == 0)
    def _():
        ssd_ref[...] = jnp.zeros_like(ssd_ref)

    ssd_ref[...] += jnp.sum(d * d, axis=0, keepdims=True)


def _tc_ssd(hp, mu):
    return pl.pallas_call(
        _ssd_body,
        grid=(NGRID,),
        in_specs=[
            pl.BlockSpec((TCBLK, H), lambda i: (i, 0)),
            pl.BlockSpec((1, H), lambda i: (0, 0)),
        ],
        out_specs=pl.BlockSpec((1, H), lambda i: (0, 0)),
        out_shape=jax.ShapeDtypeStruct((1, H), jnp.float32),
    )(hp, mu)


def _bn_body(hp_ref, mu_ref, k_ref, b_ref, o_ref):
    o_ref[...] = (hp_ref[...] - mu_ref[...]) * k_ref[...] + b_ref[...]


def _tc_bn(hp, mu, k, b):
    return pl.pallas_call(
        _bn_body,
        grid=(NGRID,),
        in_specs=[
            pl.BlockSpec((TCBLK, H), lambda i: (i, 0)),
            pl.BlockSpec((1, H), lambda i: (0, 0)),
            pl.BlockSpec((1, H), lambda i: (0, 0)),
            pl.BlockSpec((1, H), lambda i: (0, 0)),
        ],
        out_specs=pl.BlockSpec((TCBLK, H), lambda i: (i, 0)),
        out_shape=jax.ShapeDtypeStruct((NPAD, H), jnp.float32),
    )(hp, mu, k, b)


def _head_body(p0_ref, p1_ref, te_ref, fcxd_w_ref, fcxd_b_ref,
               fcxt_w_ref, fcxt_b_ref, bnxt_g_ref, bnxt_b_ref,
               fc1_w_ref, fc1_b_ref, fc2_w_ref, fc2_b_ref,
               out_w_ref, out_b_ref, o_ref):
    hg = p0_ref[...] + p1_ref[...]
    hg = jnp.maximum(
        jnp.dot(hg, fcxd_w_ref[...], preferred_element_type=jnp.float32)
        + fcxd_b_ref[...], 0.0)
    xt = (jnp.dot(te_ref[...], fcxt_w_ref[...],
                  preferred_element_type=jnp.float32) + fcxt_b_ref[...])
    mu = jnp.mean(xt, axis=0, keepdims=True)
    var = jnp.mean((xt - mu) * (xt - mu), axis=0, keepdims=True)
    xt = ((xt - mu) / jnp.sqrt(var + 1e-5) * bnxt_g_ref[...]
          + bnxt_b_ref[...])
    xt = jnp.maximum(xt, 0.0)
    xc = jnp.concatenate([hg, xt], axis=1)
    xc = jnp.maximum(
        jnp.dot(xc, fc1_w_ref[...], preferred_element_type=jnp.float32)
        + fc1_b_ref[...], 0.0)
    xc = jnp.maximum(
        jnp.dot(xc, fc2_w_ref[...], preferred_element_type=jnp.float32)
        + fc2_b_ref[...], 0.0)
    o_ref[...] = (jnp.dot(xc, out_w_ref[...],
                          preferred_element_type=jnp.float32)
                  + out_b_ref[...])


def _tc_head(p0, p1, te, fcxd_w, fcxd_b, fcxt_w, fcxt_b, bnxt_g, bnxt_b,
             fc1_w, fc1_b, fc2_w, fc2_b, out_w, out_b):
    return pl.pallas_call(
        _head_body,
        out_shape=jax.ShapeDtypeStruct((B, 1), jnp.float32),
    )(p0, p1, te, fcxd_w, fcxd_b, fcxt_w, fcxt_b, bnxt_g, bnxt_b,
      fc1_w, fc1_b, fc2_w, fc2_b, out_w, out_b)


def kernel(x, edge_index, batch, target_embedding, c1_w1, c1_b1, c1_w2,
           c1_b2, cw1, cb1, cw2, cb2, bn_g, bn_b, fcxd_w, fcxd_b, fcxt_w,
           fcxt_b, bnxt_g, bnxt_b, fc1_w, fc1_b, fc2_w, fc2_b, out_w, out_b):
    f32 = jnp.float32
    x80 = jnp.pad(x, ((0, NPAD - N), (0, FPAD - F)))
    src = jnp.pad(edge_index[0], (0, EPAD - E)).reshape(EPAD // EBLK, EBLK)
    dst = jnp.pad(edge_index[1], (0, EPAD - E),
                  constant_values=N).reshape(EPAD // EBLK, EBLK)
    bid = jnp.pad(batch, (0, NPAD - N),
                  constant_values=B).reshape(NW, P_TBLK, EBLK)
    bid = jnp.pad(bid, ((0, 0), (0, P_TPAD - P_TBLK), (0, 0)),
                  constant_values=B)
    zero32 = jnp.zeros((N_TROW, H), f32)
    zero16 = jnp.zeros((N_TROW, 16), f32)
    w1p = jnp.pad(c1_w1, ((0, FPAD - F), (0, 0)))

    agg32 = _make_sc_edge_agg(H)
    h = None
    for i in range(5):
        if i == 0:
            xa = x80[:, 0:32]
            xb = x80[:, 32:64]
            xc = x80[:, 64:80]
            pa = agg32(xa, src, dst, zero32)
            pb = agg32(xb, src, dst, zero32)
            pc = _make_sc_edge_agg(16)(xc, src, dst, zero16)
            a0 = jnp.concatenate([pa[:NPAD], pb[:NPAD], pc[:NPAD]], axis=1)
            a1 = jnp.concatenate([pa[NPAD:], pb[NPAD:], pc[NPAD:]], axis=1)
            hp, s = _tc_gin(x80, a0, a1, w1p, c1_b1[None, :], c1_w2,
                            c1_b2[None, :])
        else:
            parts = agg32(h, src, dst, zero32)
            hp, s = _tc_gin(h, parts[:NPAD], parts[NPAD:], cw1[i - 1],
                            cb1[i - 1][None, :], cw2[i - 1],
                            cb2[i - 1][None, :])
        mu = s / N
        ssd = _tc_ssd(hp, mu)
        var = ssd / N
        k = bn_g[i][None, :] / jnp.sqrt(var + 1e-5)
        h = _tc_bn(hp, mu, k, bn_b[i][None, :])

    pool = _make_sc_pool()(h, bid, zero32)
    p0 = pool[:B]
    p1 = pool[BACC:BACC + B]
    return _tc_head(p0, p1, target_embedding, fcxd_w, fcxd_b[None, :],
                    fcxt_w, fcxt_b[None, :], bnxt_g[None, :], bnxt_b[None, :],
                    fc1_w, fc1_b[None, :], fc2_w, fc2_b[None, :],
                    out_w, out_b[None, :])


# speedup vs baseline: 4.0544x; 1.2107x over previous
import functools

import jax
import jax.numpy as jnp
from jax import lax
from jax.experimental import pallas as pl
from jax.experimental.pallas import tpu as pltpu
from jax.experimental.pallas import tpu_sc as plsc

N = 50000
E = 800000
B = 256
H = 32
F = 78
FPAD = 80

NC = 2
NS = 16
NW = NC * NS

EBLK = 128
ECH = 8
NPAD = 53248
R_TILE = NPAD // NW
R_PAD = R_TILE + 8
EBLKS = (E + EBLK - 1) // EBLK + ECH
EPAD2 = EBLKS * EBLK
P_TBLK = NPAD // (NW * EBLK)
P_TPAD = 16
BACC = 384
B_TROW = BACC // NS

TCBLK = 512
NGRID = NPAD // TCBLK


def _sc_mesh():
    return plsc.VectorSubcoreMesh(core_axis_name="c", subcore_axis_name="s",
                                  num_cores=NC, num_subcores=NS)


@functools.cache
def _make_sc_edge_agg(w):
  @functools.partial(
    pl.kernel,
    out_type=jax.ShapeDtypeStruct((NPAD, w), jnp.float32),
    mesh=_sc_mesh(),
    scratch_types=[
        pltpu.VMEM((NW,), jnp.int32),
        pltpu.VMEM((NW,), jnp.int32),
        pltpu.VMEM((ECH, EBLK), jnp.int32),
        pltpu.VMEM((ECH, EBLK), jnp.int32),
        pltpu.VMEM((ECH, EBLK), jnp.int32),
        pltpu.VMEM((EBLK, w), jnp.float32),
        pltpu.VMEM_SHARED((NS * R_PAD, w), jnp.float32),
        pltpu.SemaphoreType.DMA,
    ],
    compiler_params=pltpu.CompilerParams(use_tc_tiling_on_sc=False,
                                         needs_layout_passes=False),
  )
  def _sc_edge_agg(y_hbm, src_hbm, dst_hbm, blk0_hbm, nch_hbm, zero_hbm,
                   out_hbm, blk0_v, nch_v, src_v, dst_v, loc_v, rows_v,
                   acc_sh, sem):
    cid = lax.axis_index("c")
    sid = lax.axis_index("s")
    gid = cid * NS + sid
    base_row = gid * R_TILE
    region = sid * R_PAD

    pltpu.sync_copy(zero_hbm, acc_sh.at[pl.ds(region, R_PAD)])

    pltpu.sync_copy(blk0_hbm, blk0_v)
    pltpu.sync_copy(nch_hbm, nch_v)
    lanes = lax.iota(jnp.int32, 16)

    def extract(v):
        lo = v[pl.ds(0, 16)]
        hi = v[pl.ds(16, 16)]
        x = lo + (hi - lo) * cid
        return jnp.max(jnp.where(lanes == sid, x, 0))

    b0 = extract(blk0_v)
    nch = extract(nch_v)

    def chunk(c, carry):
        start = b0 + c * ECH
        pltpu.sync_copy(src_hbm.at[pl.ds(start, ECH)], src_v)
        pltpu.sync_copy(dst_hbm.at[pl.ds(start, ECH)], dst_v)
        for j in range(ECH):
            for k in range(EBLK // 16):
                dv = dst_v[j, pl.ds(k * 16, 16)]
                lv = dv - base_row
                ok = (lv >= 0) & (lv < R_TILE)
                loc_v[j, pl.ds(k * 16, 16)] = (
                    jnp.where(ok, lv, R_TILE) + region)
        for j in range(ECH):
            pltpu.async_copy(y_hbm.at[src_v.at[j]], rows_v, sem).wait()
            pltpu.sync_copy(rows_v, acc_sh.at[loc_v.at[j]], add=True)
        return carry

    lax.fori_loop(0, nch, chunk, 0)

    pltpu.sync_copy(acc_sh.at[pl.ds(region, R_TILE)],
                    out_hbm.at[pl.ds(base_row, R_TILE)])

  return _sc_edge_agg


@functools.cache
def _make_sc_pool():
  @functools.partial(
    pl.kernel,
    out_type=jax.ShapeDtypeStruct((NC * BACC, H), jnp.float32),
    mesh=_sc_mesh(),
    scratch_types=[
        pltpu.VMEM((P_TPAD, EBLK), jnp.int32),
        pltpu.VMEM((EBLK, H), jnp.float32),
        pltpu.VMEM_SHARED((BACC, H), jnp.float32),
        pltpu.SemaphoreType.DMA,
    ],
    compiler_params=pltpu.CompilerParams(use_tc_tiling_on_sc=False),
  )
  def _sc_pool(h_hbm, bid_hbm, zero_hbm, out_hbm, bid_v, rows_v, acc_sh, sem):
    cid = lax.axis_index("c")
    sid = lax.axis_index("s")
    wid = sid * NC + cid

    pltpu.sync_copy(zero_hbm.at[pl.ds(0, B_TROW)],
                    acc_sh.at[pl.ds(sid * B_TROW, B_TROW)])
    pltpu.sync_copy(bid_hbm.at[wid], bid_v)
    plsc.subcore_barrier()

    def body(j, carry):
        pltpu.async_copy(
            h_hbm.at[pl.ds((wid * P_TBLK + j) * EBLK, EBLK)], rows_v,
            sem).wait()
        pltpu.sync_copy(rows_v, acc_sh.at[bid_v.at[j]], add=True)
        return carry

    lax.fori_loop(0, P_TBLK, body, 0)
    plsc.subcore_barrier()

    pltpu.sync_copy(acc_sh.at[pl.ds(sid * B_TROW, B_TROW)],
                    out_hbm.at[pl.ds(cid * BACC + sid * B_TROW, B_TROW)])

  return _sc_pool


def _mask_rows(i):
    return i * TCBLK + lax.broadcasted_iota(jnp.int32, (TCBLK, 1), 0)


def _gin_tail(i, m, w1_ref, b1_ref, w2_ref, b2_ref, hp_ref, s_ref):
    u = jnp.maximum(
        jnp.dot(m, w1_ref[...], preferred_element_type=jnp.float32)
        + b1_ref[...], 0.0)
    hp = jnp.maximum(
        jnp.dot(u, w2_ref[...], preferred_element_type=jnp.float32)
        + b2_ref[...], 0.0)
    hp_ref[...] = hp
    hm = jnp.where(_mask_rows(i) < N, hp, 0.0)

    @pl.when(i == 0)
    def _():
        s_ref[...] = jnp.zeros_like(s_ref)

    s_ref[...] += jnp.sum(hm, axis=0, keepdims=True)


def _gin_body(h_ref, a_ref, w1_ref, b1_ref, w2_ref, b2_ref, hp_ref, s_ref):
    i = pl.program_id(0)
    m = h_ref[...] + a_ref[...]
    _gin_tail(i, m, w1_ref, b1_ref, w2_ref, b2_ref, hp_ref, s_ref)


def _gin1_body(h_ref, aa_ref, ab_ref, ac_ref, w1_ref, b1_ref, w2_ref,
               b2_ref, hp_ref, s_ref):
    i = pl.program_id(0)
    agg = jnp.concatenate([aa_ref[...], ab_ref[...], ac_ref[...]], axis=1)
    m = h_ref[...] + agg
    _gin_tail(i, m, w1_ref, b1_ref, w2_ref, b2_ref, hp_ref, s_ref)


def _gin_specs(fin, n_agg, widths):
    in_specs = [pl.BlockSpec((TCBLK, fin), lambda i: (i, 0))]
    for wdt in widths:
        in_specs.append(
            pl.BlockSpec((TCBLK, wdt), lambda i: (i, 0)))
    in_specs += [
        pl.BlockSpec((fin, H), lambda i: (0, 0)),
        pl.BlockSpec((1, H), lambda i: (0, 0)),
        pl.BlockSpec((H, H), lambda i: (0, 0)),
        pl.BlockSpec((1, H), lambda i: (0, 0)),
    ]
    return in_specs


def _gin_outs():
    return (
        [pl.BlockSpec((TCBLK, H), lambda i: (i, 0)),
         pl.BlockSpec((1, H), lambda i: (0, 0))],
        [jax.ShapeDtypeStruct((NPAD, H), jnp.float32),
         jax.ShapeDtypeStruct((1, H), jnp.float32)],
    )


def _tc_gin(h, a, w1, b1, w2, b2):
    out_specs, out_shape = _gin_outs()
    return pl.pallas_call(
        _gin_body, grid=(NGRID,),
        in_specs=_gin_specs(H, 1, [H]),
        out_specs=out_specs, out_shape=out_shape,
    )(h, a, w1, b1, w2, b2)


def _tc_gin1(h, aa, ab, ac, w1, b1, w2, b2):
    out_specs, out_shape = _gin_outs()
    return pl.pallas_call(
        _gin1_body, grid=(NGRID,),
        in_specs=_gin_specs(FPAD, 3, [H, H, 16]),
        out_specs=out_specs, out_shape=out_shape,
    )(h, aa, ab, ac, w1, b1, w2, b2)


def _ssd_body(hp_ref, mu_ref, ssd_ref):
    i = pl.program_id(0)
    d = hp_ref[...] - mu_ref[...]
    d = jnp.where(_mask_rows(i) < N, d, 0.0)

    @pl.when(i == 0)
    def _():
        ssd_ref[...] = jnp.zeros_like(ssd_ref)

    ssd_ref[...] += jnp.sum(d * d, axis=0, keepdims=True)


def _tc_ssd(hp, mu):
    return pl.pallas_call(
        _ssd_body,
        grid=(NGRID,),
        in_specs=[
            pl.BlockSpec((TCBLK, H), lambda i: (i, 0)),
            pl.BlockSpec((1, H), lambda i: (0, 0)),
        ],
        out_specs=pl.BlockSpec((1, H), lambda i: (0, 0)),
        out_shape=jax.ShapeDtypeStruct((1, H), jnp.float32),
    )(hp, mu)


def _bn_body(hp_ref, mu_ref, var_ref, g_ref, b_ref, o_ref):
    o_ref[...] = ((hp_ref[...] - mu_ref[...])
                  / jnp.sqrt(var_ref[...] + 1e-5) * g_ref[...] + b_ref[...])


def _tc_bn(hp, mu, var, g, b):
    return pl.pallas_call(
        _bn_body,
        grid=(NGRID,),
        in_specs=[
            pl.BlockSpec((TCBLK, H), lambda i: (i, 0)),
            pl.BlockSpec((1, H), lambda i: (0, 0)),
            pl.BlockSpec((1, H), lambda i: (0, 0)),
            pl.BlockSpec((1, H), lambda i: (0, 0)),
            pl.BlockSpec((1, H), lambda i: (0, 0)),
        ],
        out_specs=pl.BlockSpec((TCBLK, H), lambda i: (i, 0)),
        out_shape=jax.ShapeDtypeStruct((NPAD, H), jnp.float32),
    )(hp, mu, var, g, b)


def _head_body(p0_ref, p1_ref, te_ref, fcxd_w_ref, fcxd_b_ref,
               fcxt_w_ref, fcxt_b_ref, bnxt_g_ref, bnxt_b_ref,
               fc1_w_ref, fc1_b_ref, fc2_w_ref, fc2_b_ref,
               out_w_ref, out_b_ref, o_ref):
    hg = p0_ref[...] + p1_ref[...]
    hg = jnp.maximum(
        jnp.dot(hg, fcxd_w_ref[...], preferred_element_type=jnp.float32)
        + fcxd_b_ref[...], 0.0)
    xt = (jnp.dot(te_ref[...], fcxt_w_ref[...],
                  preferred_element_type=jnp.float32) + fcxt_b_ref[...])
    mu = jnp.mean(xt, axis=0, keepdims=True)
    var = jnp.mean((xt - mu) * (xt - mu), axis=0, keepdims=True)
    xt = ((xt - mu) / jnp.sqrt(var + 1e-5) * bnxt_g_ref[...]
          + bnxt_b_ref[...])
    xt = jnp.maximum(xt, 0.0)
    xc = jnp.concatenate([hg, xt], axis=1)
    xc = jnp.maximum(
        jnp.dot(xc, fc1_w_ref[...], preferred_element_type=jnp.float32)
        + fc1_b_ref[...], 0.0)
    xc = jnp.maximum(
        jnp.dot(xc, fc2_w_ref[...], preferred_element_type=jnp.float32)
        + fc2_b_ref[...], 0.0)
    o_ref[...] = (jnp.dot(xc, out_w_ref[...],
                          preferred_element_type=jnp.float32)
                  + out_b_ref[...])


def _tc_head(p0, p1, te, fcxd_w, fcxd_b, fcxt_w, fcxt_b, bnxt_g, bnxt_b,
             fc1_w, fc1_b, fc2_w, fc2_b, out_w, out_b):
    return pl.pallas_call(
        _head_body,
        out_shape=jax.ShapeDtypeStruct((B, 1), jnp.float32),
    )(p0, p1, te, fcxd_w, fcxd_b, fcxt_w, fcxt_b, bnxt_g, bnxt_b,
      fc1_w, fc1_b, fc2_w, fc2_b, out_w, out_b)


def kernel(x, edge_index, batch, target_embedding, c1_w1, c1_b1, c1_w2,
           c1_b2, cw1, cb1, cw2, cb2, bn_g, bn_b, fcxd_w, fcxd_b, fcxt_w,
           fcxt_b, bnxt_g, bnxt_b, fc1_w, fc1_b, fc2_w, fc2_b, out_w, out_b):
    f32 = jnp.float32
    x80 = jnp.pad(x, ((0, NPAD - N), (0, FPAD - F)))

    order = jnp.argsort(edge_index[1], stable=True)
    srcs = jnp.pad(edge_index[0][order], (0, EPAD2 - E))
    dsts_flat = jnp.pad(edge_index[1][order], (0, EPAD2 - E),
                        constant_values=N)
    srcs = srcs.reshape(EBLKS, EBLK)
    dsts = dsts_flat.reshape(EBLKS, EBLK)
    bounds = jnp.searchsorted(
        dsts_flat, (jnp.arange(NW + 1) * R_TILE).astype(jnp.int32)
    ).astype(jnp.int32)
    blk0 = bounds[:NW] // EBLK
    blk1 = (bounds[1:] + EBLK - 1) // EBLK
    nch = (jnp.maximum(blk1 - blk0, 0) + ECH - 1) // ECH

    bid = jnp.pad(batch, (0, NPAD - N),
                  constant_values=B).reshape(NW, P_TBLK, EBLK)
    bid = jnp.pad(bid, ((0, 0), (0, P_TPAD - P_TBLK), (0, 0)),
                  constant_values=B)
    zero32 = jnp.zeros((R_PAD, H), f32)
    zero16 = jnp.zeros((R_PAD, 16), f32)
    w1p = jnp.pad(c1_w1, ((0, FPAD - F), (0, 0)))

    agg32 = _make_sc_edge_agg(H)
    h = None
    for i in range(5):
        if i == 0:
            aa = agg32(x80[:, 0:32], srcs, dsts, blk0, nch, zero32)
            ab = agg32(x80[:, 32:64], srcs, dsts, blk0, nch, zero32)
            ac = _make_sc_edge_agg(16)(x80[:, 64:80], srcs, dsts, blk0,
                                       nch, zero16)
            hp, s = _tc_gin1(x80, aa, ab, ac, w1p, c1_b1[None, :], c1_w2,
                             c1_b2[None, :])
        else:
            a = agg32(h, srcs, dsts, blk0, nch, zero32)
            hp, s = _tc_gin(h, a, cw1[i - 1], cb1[i - 1][None, :],
                            cw2[i - 1], cb2[i - 1][None, :])
        mu = jnp.mean(hp[:N], axis=0)
        var = jnp.mean((hp[:N] - mu) ** 2, axis=0)
        h = _tc_bn(hp, mu[None, :], var[None, :], bn_g[i][None, :],
                   bn_b[i][None, :])

    pool = _make_sc_pool()(h, bid, zero32[:B_TROW])
    p0 = pool[:B]
    p1 = pool[BACC:BACC + B]
    return _tc_head(p0, p1, target_embedding, fcxd_w, fcxd_b[None, :],
                    fcxt_w, fcxt_b[None, :], bnxt_g[None, :], bnxt_b[None, :],
                    fc1_w, fc1_b[None, :], fc2_w, fc2_b[None, :],
                    out_w, out_b[None, :])
